# baseline (speedup 1.0000x reference)
import functools

import jax
import jax.numpy as jnp
from jax import lax
from jax.experimental import pallas as pl
from jax.experimental.pallas import tpu as pltpu
from jax.experimental.pallas import tpu_sc as plsc

B, S, DIN, D, HID, E, NCLS = 64, 196, 768, 64, 256, 4, 10
T = B * S
BT = 784
NBLK = T // BT
BTC = 512
NBC = T // BTC + E
TP = NBC * BTC
NW = 32
CH = T // NW


def _gelu(v):
    return 0.5 * v * (1.0 + lax.erf(v * 0.7071067811865476))


def _encode_route_body(x_ref, pe_w1_ref, pe_b1_ref, pe_w2_ref, pe_b2_ref,
                       gate_w_ref, gate_b_ref,
                       tok_ref, gate_ref, idx_ref, rank_ref,
                       cnt_ref, imp_ref,
                       cnt_acc, imp_acc):
    i = pl.program_id(0)

    @pl.when(i == 0)
    def _():
        cnt_acc[...] = jnp.zeros_like(cnt_acc)
        imp_acc[...] = jnp.zeros_like(imp_acc)

    xb = x_ref[...]
    h1 = _gelu(jnp.dot(xb, pe_w1_ref[...], preferred_element_type=jnp.float32)
               + pe_b1_ref[...])
    tok = (jnp.dot(h1, pe_w2_ref[...], preferred_element_type=jnp.float32)
           + pe_b2_ref[...])
    tok_ref[...] = jnp.concatenate(
        [tok, jnp.zeros((BT, 128 - D), jnp.float32)], axis=1)

    glog = (jnp.dot(tok, gate_w_ref[...], preferred_element_type=jnp.float32)
            + gate_b_ref[...])
    m = jnp.max(glog, axis=-1, keepdims=True)
    p = jnp.exp(glog - m)
    probs = p / jnp.sum(p, axis=-1, keepdims=True)
    gval = jnp.max(probs, axis=-1, keepdims=True)
    idx = jnp.argmax(probs, axis=-1).reshape(BT, 1)
    eids = lax.broadcasted_iota(jnp.int32, (BT, E), 1)
    oh = (idx == eids).astype(jnp.float32)

    cs = oh
    k = 1
    while k < BT:
        cs = cs + jnp.concatenate(
            [jnp.zeros((k, E), jnp.float32), cs[:-k, :]], axis=0)
        k *= 2
    excl = cs - oh
    rank = jnp.sum(oh * (excl + cnt_acc[...]), axis=1,
                   keepdims=True).astype(jnp.int32)

    gate_ref[...] = gval
    idx_ref[...] = idx
    rank_ref[...] = rank

    cnt_acc[...] += cs[BT - 1:BT, :]
    imp_acc[...] += jnp.sum(probs, axis=0, keepdims=True)

    @pl.when(i == NBLK - 1)
    def _():
        zpad = jnp.zeros((1, 128 - E), jnp.float32)
        cnt_ref[...] = jnp.concatenate([cnt_acc[...], zpad], axis=1)
        imp_ref[...] = jnp.concatenate([imp_acc[...], zpad], axis=1)


def _encode_route(xr, pe_w1, pe_b1, pe_w2, pe_b2, gate_w, gate_b,
                  interpret=False):
    full = lambda shape: pl.BlockSpec(shape, lambda i: (0,) * len(shape))
    return pl.pallas_call(
        _encode_route_body,
        grid=(NBLK,),
        in_specs=[
            pl.BlockSpec((BT, DIN), lambda i: (i, 0)),
            full((DIN, D)),
            full((1, D)),
            full((D, D)),
            full((1, D)),
            full((D, E)),
            full((1, E)),
        ],
        out_specs=[
            pl.BlockSpec((BT, 128), lambda i: (i, 0)),
            pl.BlockSpec((BT, 1), lambda i: (i, 0)),
            pl.BlockSpec((BT, 1), lambda i: (i, 0)),
            pl.BlockSpec((BT, 1), lambda i: (i, 0)),
            pl.BlockSpec((1, 128), lambda i: (0, 0)),
            pl.BlockSpec((1, 128), lambda i: (0, 0)),
        ],
        out_shape=[
            jax.ShapeDtypeStruct((T, 128), jnp.float32),
            jax.ShapeDtypeStruct((T, 1), jnp.float32),
            jax.ShapeDtypeStruct((T, 1), jnp.int32),
            jax.ShapeDtypeStruct((T, 1), jnp.int32),
            jax.ShapeDtypeStruct((1, 128), jnp.float32),
            jax.ShapeDtypeStruct((1, 128), jnp.float32),
        ],
        scratch_shapes=[
            pltpu.VMEM((1, E), jnp.float32),
            pltpu.VMEM((1, E), jnp.float32),
        ],
        interpret=interpret,
    )(xr, pe_w1, pe_b1.reshape(1, D), pe_w2, pe_b2.reshape(1, D),
      gate_w, gate_b.reshape(1, E))


def _dest_body(idx_ref, rank_ref, off_ref, dest_ref):
    idxv = idx_ref[...]
    d = rank_ref[...]
    for e in range(E):
        d = d + jnp.where(idxv == e, off_ref[0:1, e:e + 1], 0)
    dest_ref[...] = d


def _dest_slots(idx2, rank2, off, interpret=False):
    full = lambda shape: pl.BlockSpec(shape, lambda: (0,) * len(shape))
    return pl.pallas_call(
        _dest_body,
        in_specs=[full((T // 128, 128)), full((T // 128, 128)),
                  full((1, 128))],
        out_specs=full((T // 128, 128)),
        out_shape=jax.ShapeDtypeStruct((T // 128, 128), jnp.int32),
        interpret=interpret,
    )(idx2, rank2, off)


CHS = T // 16
SL = TP // 16
CHP = TP // NW


def _sc_build_perm(dest, tids, zeros_tp):
    mesh = plsc.VectorSubcoreMesh(core_axis_name="c", subcore_axis_name="s")

    @functools.partial(
        pl.kernel, mesh=mesh,
        out_type=jax.ShapeDtypeStruct((TP,), jnp.int32),
        scratch_types=[
            pltpu.VMEM((CHS,), jnp.int32),
            pltpu.VMEM((CHS,), jnp.int32),
            pltpu.VMEM_SHARED((TP,), jnp.int32),
        ],
    )
    def build(dest_hbm, tid_hbm, zero_hbm, perm_hbm, dest_v, tid_v, shared):
        cid = lax.axis_index("c")
        sid = lax.axis_index("s")

        @pl.when(cid == 0)
        def _():
            sbase = sid * SL
            pltpu.sync_copy(zero_hbm.at[pl.ds(sbase, SL)],
                            shared.at[pl.ds(sbase, SL)])
            base = sid * CHS
            pltpu.sync_copy(dest_hbm.at[pl.ds(base, CHS)], dest_v)
            pltpu.sync_copy(tid_hbm.at[pl.ds(base, CHS)], tid_v)
            plsc.subcore_barrier()
            pltpu.sync_copy(tid_v, shared.at[dest_v])
            plsc.subcore_barrier()
            pltpu.sync_copy(shared.at[pl.ds(sbase, SL)],
                            perm_hbm.at[pl.ds(sbase, SL)])

    return build(dest, tids, zeros_tp)


def _sc_gather_rows(table, indices, n_rows):
    mesh = plsc.VectorSubcoreMesh(core_axis_name="c", subcore_axis_name="s")
    chunk = n_rows // NW

    @functools.partial(
        pl.kernel, mesh=mesh,
        out_type=jax.ShapeDtypeStruct((n_rows, 128), jnp.float32),
        scratch_types=[
            pltpu.VMEM((chunk,), jnp.int32),
            pltpu.VMEM((chunk, 128), jnp.float32),
            pltpu.SemaphoreType.DMA,
        ],
    )
    def gat(table_hbm, idx_hbm, out_hbm, idx_v, rows_v, sem):
        wid = lax.axis_index("s") * 2 + lax.axis_index("c")
        base = wid * chunk
        pltpu.sync_copy(idx_hbm.at[pl.ds(base, chunk)], idx_v)
        pltpu.async_copy(table_hbm.at[idx_v], rows_v, sem).wait()
        pltpu.sync_copy(rows_v, out_hbm.at[pl.ds(base, chunk)])

    return gat(table, indices)


def _expert_body(be_ref, tok_ref, w1_ref, b1_ref, w2_ref, b2_ref, out_ref):
    tok = tok_ref[:, :D]
    he = _gelu(jnp.dot(tok, w1_ref[0], preferred_element_type=jnp.float32)
               + b1_ref[0])
    out = (jnp.dot(he, w2_ref[0], preferred_element_type=jnp.float32)
           + b2_ref[0])
    out_ref[...] = jnp.concatenate(
        [out, jnp.zeros((BTC, 128 - D), jnp.float32)], axis=1)


def _expert_ffn(block_expert, sorted_tok, exp_w1, exp_b1, exp_w2, exp_b2,
                interpret=False):
    grid_spec = pltpu.PrefetchScalarGridSpec(
        num_scalar_prefetch=1,
        grid=(NBC,),
        in_specs=[
            pl.BlockSpec((BTC, 128), lambda i, be: (i, 0)),
            pl.BlockSpec((1, D, HID), lambda i, be: (be[i], 0, 0)),
            pl.BlockSpec((1, 1, HID), lambda i, be: (be[i], 0, 0)),
            pl.BlockSpec((1, HID, D), lambda i, be: (be[i], 0, 0)),
            pl.BlockSpec((1, 1, D), lambda i, be: (be[i], 0, 0)),
        ],
        out_specs=pl.BlockSpec((BTC, 128), lambda i, be: (i, 0)),
    )
    return pl.pallas_call(
        _expert_body,
        grid_spec=grid_spec,
        out_shape=jax.ShapeDtypeStruct((TP, 128), jnp.float32),
        interpret=interpret,
    )(block_expert, sorted_tok, exp_w1, exp_b1.reshape(E, 1, HID),
      exp_w2, exp_b2.reshape(E, 1, D))


BBE = 8
NBE = B // BBE


def _tail_body(rows_ref, gate_ref, cnt_ref, imp_ref, ln_g_ref, ln_b_ref,
               cls_w1_ref, cls_b1_ref, cls_w2_ref, cls_b2_ref,
               moe_ref, logits_ref, aux_ref, pooled_acc):
    i = pl.program_id(0)
    rows = rows_ref[..., :D]
    g = gate_ref[...]
    mo = rows * g[:, :, None]
    moe_ref[...] = mo
    pooled_acc[pl.ds(i * BBE, BBE), :] = jnp.mean(mo, axis=1)

    @pl.when(i == NBE - 1)
    def _():
        pooled = pooled_acc[...]
        mu = jnp.mean(pooled, axis=-1, keepdims=True)
        var = jnp.mean((pooled - mu) ** 2, axis=-1, keepdims=True)
        ln = ((pooled - mu) / jnp.sqrt(var + 1e-5) * ln_g_ref[...]
              + ln_b_ref[...])
        c = _gelu(jnp.dot(ln, cls_w1_ref[...],
                          preferred_element_type=jnp.float32)
                  + cls_b1_ref[...])
        logits_ref[...] = (jnp.dot(c, cls_w2_ref[...],
                                   preferred_element_type=jnp.float32)
                           + cls_b2_ref[...])
        aux = (E / (T * T)) * jnp.sum(imp_ref[...] * cnt_ref[...])
        aux_ref[...] = jnp.full((1, 128), aux, dtype=jnp.float32)


def _tail(rows3, gate2, cnt, imp, ln_g, ln_b, cls_w1, cls_b1, cls_w2, cls_b2,
          interpret=False):
    full = lambda shape: pl.BlockSpec(shape, lambda i: (0,) * len(shape))
    return pl.pallas_call(
        _tail_body,
        grid=(NBE,),
        in_specs=[
            pl.BlockSpec((BBE, S, 128), lambda i: (i, 0, 0)),
            pl.BlockSpec((BBE, S), lambda i: (i, 0)),
            full((1, 128)),
            full((1, 128)),
            full((1, D)),
            full((1, D)),
            full((D, D)),
            full((1, D)),
            full((D, NCLS)),
            full((1, NCLS)),
        ],
        out_specs=[
            pl.BlockSpec((BBE, S, D), lambda i: (i, 0, 0)),
            pl.BlockSpec((B, NCLS), lambda i: (0, 0)),
            pl.BlockSpec((1, 128), lambda i: (0, 0)),
        ],
        out_shape=[
            jax.ShapeDtypeStruct((B, S, D), jnp.float32),
            jax.ShapeDtypeStruct((B, NCLS), jnp.float32),
            jax.ShapeDtypeStruct((1, 128), jnp.float32),
        ],
        scratch_shapes=[pltpu.VMEM((B, D), jnp.float32)],
        interpret=interpret,
    )(rows3, gate2, cnt, imp, ln_g.reshape(1, D), ln_b.reshape(1, D),
      cls_w1, cls_b1.reshape(1, D), cls_w2, cls_b2.reshape(1, NCLS))


def _routing_maps(cnt):
    counts = cnt[0, :E].astype(jnp.int32)
    pblocks = (counts + BTC - 1) // BTC
    csb = jnp.cumsum(pblocks)
    starts_blk = jnp.concatenate(
        [jnp.zeros((1,), jnp.int32), csb[:-1].astype(jnp.int32)])
    off = jnp.concatenate(
        [(starts_blk * BTC).astype(jnp.int32),
         jnp.zeros((128 - E,), jnp.int32)]).reshape(1, 128)
    block_expert = jnp.clip(
        jnp.searchsorted(csb, jnp.arange(NBC), side='right'),
        0, E - 1).astype(jnp.int32)
    return off, block_expert


def kernel(x, pe_w1, pe_b1, pe_w2, pe_b2, gate_w, gate_b, exp_w1, exp_b1,
           exp_w2, exp_b2, ln_g, ln_b, cls_w1, cls_b1, cls_w2, cls_b2):
    xr = x.reshape(T, DIN)
    tokens, gate, idx, rank, cnt, imp = _encode_route(
        xr, pe_w1, pe_b1, pe_w2, pe_b2, gate_w, gate_b)

    off, block_expert = _routing_maps(cnt)
    dest2 = _dest_slots(idx.reshape(T // 128, 128),
                        rank.reshape(T // 128, 128), off)
    dest = dest2.reshape(T)
    perm = _sc_build_perm(dest, jnp.arange(T, dtype=jnp.int32),
                          jnp.zeros((TP,), jnp.int32))
    sorted_tok = _sc_gather_rows(tokens, perm, TP)
    sorted_moe = _expert_ffn(
        block_expert, sorted_tok, exp_w1, exp_b1, exp_w2, exp_b2)
    rows = _sc_gather_rows(sorted_moe, dest, T)

    moe_out, logits, aux = _tail(
        rows.reshape(B, S, 128), gate.reshape(B, S), cnt, imp,
        ln_g, ln_b, cls_w1, cls_b1, cls_w2, cls_b2)
    return logits, moe_out, aux[0, 0]


# speedup vs baseline: 1.7379x; 1.7379x over previous
import functools

import jax
import jax.numpy as jnp
from jax.experimental import pallas as pl
from jax.experimental.pallas import tpu as pltpu

B, S, DIN, D, HID, E, NCLS = 64, 196, 768, 64, 256, 4, 10
T = B * S
BT = 784
NBLK = T // BT
BATCHES_PER_BLK = BT // S


def _gelu(v):
    return 0.5 * v * (1.0 + jax.lax.erf(v * 0.7071067811865476))


def _fused_body(x_ref, pe_w1_ref, pe_b1_ref, pe_w2_ref, pe_b2_ref,
                gate_w_ref, gate_b_ref, exp_w1_ref, exp_b1_ref,
                exp_w2_ref, exp_b2_ref, ln_g_ref, ln_b_ref,
                cls_w1_ref, cls_b1_ref, cls_w2_ref, cls_b2_ref,
                logits_ref, moe_ref, aux_ref,
                pooled_acc, imp_acc, cnt_acc):
    i = pl.program_id(0)

    @pl.when(i == 0)
    def _():
        imp_acc[...] = jnp.zeros_like(imp_acc)
        cnt_acc[...] = jnp.zeros_like(cnt_acc)

    xb = x_ref[...]
    h1 = _gelu(jnp.dot(xb, pe_w1_ref[...], preferred_element_type=jnp.float32)
               + pe_b1_ref[...])
    tok = (jnp.dot(h1, pe_w2_ref[...], preferred_element_type=jnp.float32)
           + pe_b2_ref[...])

    glog = (jnp.dot(tok, gate_w_ref[...], preferred_element_type=jnp.float32)
            + gate_b_ref[...])
    m = jnp.max(glog, axis=-1, keepdims=True)
    p = jnp.exp(glog - m)
    probs = p / jnp.sum(p, axis=-1, keepdims=True)
    gval = jnp.max(probs, axis=-1, keepdims=True)
    idx = jnp.argmax(probs, axis=-1).reshape(BT, 1)
    eids = jax.lax.broadcasted_iota(jnp.int32, (BT, E), 1)
    oh = (idx == eids).astype(jnp.float32)

    imp_acc[...] += jnp.sum(probs, axis=0, keepdims=True)
    cnt_acc[...] += jnp.sum(oh, axis=0, keepdims=True)

    combine = oh * gval
    he_all = _gelu(
        jnp.dot(tok.astype(jnp.bfloat16), exp_w1_ref[...],
                preferred_element_type=jnp.float32)
        + exp_b1_ref[...])
    scaled = (he_all.reshape(BT, E, HID)
              * combine[:, :, None]).reshape(BT, E * HID)
    moe = (jnp.dot(scaled.astype(jnp.bfloat16), exp_w2_ref[...],
                   preferred_element_type=jnp.float32)
           + jnp.dot(combine, exp_b2_ref[...],
                     preferred_element_type=jnp.float32))

    moe3 = moe.reshape(BATCHES_PER_BLK, S, D)
    moe_ref[...] = moe3
    pooled_acc[pl.ds(i * BATCHES_PER_BLK, BATCHES_PER_BLK), :] = (
        jnp.mean(moe3, axis=1))

    @pl.when(i == NBLK - 1)
    def _():
        pooled = pooled_acc[...]
        mu = jnp.mean(pooled, axis=-1, keepdims=True)
        var = jnp.mean((pooled - mu) ** 2, axis=-1, keepdims=True)
        ln = ((pooled - mu) / jnp.sqrt(var + 1e-5) * ln_g_ref[...]
              + ln_b_ref[...])
        c = _gelu(jnp.dot(ln, cls_w1_ref[...],
                          preferred_element_type=jnp.float32)
                  + cls_b1_ref[...])
        logits_ref[...] = (jnp.dot(c, cls_w2_ref[...],
                                   preferred_element_type=jnp.float32)
                           + cls_b2_ref[...])
        imp = imp_acc[...] / T
        load = cnt_acc[...] / T
        aux = E * jnp.sum(imp * load)
        aux_ref[...] = jnp.full((1, 128), aux, dtype=jnp.float32)


def kernel(x, pe_w1, pe_b1, pe_w2, pe_b2, gate_w, gate_b, exp_w1, exp_b1,
           exp_w2, exp_b2, ln_g, ln_b, cls_w1, cls_b1, cls_w2, cls_b2,
           interpret=False):
    xr = x.reshape(T, DIN)

    full = lambda shape: pl.BlockSpec(shape, lambda i: (0,) * len(shape))
    logits, moe_out, aux = pl.pallas_call(
        _fused_body,
        grid=(NBLK,),
        in_specs=[
            pl.BlockSpec((BT, DIN), lambda i: (i, 0)),
            full((DIN, D)),
            full((1, D)),
            full((D, D)),
            full((1, D)),
            full((D, E)),
            full((1, E)),
            full((D, E * HID)),
            full((1, E * HID)),
            full((E * HID, D)),
            full((E, D)),
            full((1, D)),
            full((1, D)),
            full((D, D)),
            full((1, D)),
            full((D, NCLS)),
            full((1, NCLS)),
        ],
        out_specs=[
            pl.BlockSpec((B, NCLS), lambda i: (0, 0)),
            pl.BlockSpec((BATCHES_PER_BLK, S, D), lambda i: (i, 0, 0)),
            pl.BlockSpec((1, 128), lambda i: (0, 0)),
        ],
        out_shape=[
            jax.ShapeDtypeStruct((B, NCLS), jnp.float32),
            jax.ShapeDtypeStruct((B, S, D), jnp.float32),
            jax.ShapeDtypeStruct((1, 128), jnp.float32),
        ],
        scratch_shapes=[
            pltpu.VMEM((B, D), jnp.float32),
            pltpu.VMEM((1, E), jnp.float32),
            pltpu.VMEM((1, E), jnp.float32),
        ],
        interpret=interpret,
    )(xr, pe_w1, pe_b1.reshape(1, D), pe_w2, pe_b2.reshape(1, D),
      gate_w, gate_b.reshape(1, E),
      exp_w1.transpose(1, 0, 2).reshape(D, E * HID).astype(jnp.bfloat16),
      exp_b1.reshape(1, E * HID),
      exp_w2.reshape(E * HID, D).astype(jnp.bfloat16),
      exp_b2,
      ln_g.reshape(1, D), ln_b.reshape(1, D), cls_w1,
      cls_b1.reshape(1, D), cls_w2, cls_b2.reshape(1, NCLS))
    return logits, moe_out, aux[0, 0]


# speedup vs baseline: 2.0151x; 1.1595x over previous
import functools

import jax
import jax.numpy as jnp
from jax.experimental import pallas as pl
from jax.experimental.pallas import tpu as pltpu

B, S, DIN, D, HID, E, NCLS = 64, 196, 768, 64, 256, 4, 10
T = B * S
BT = 784
NBLK = T // BT
BATCHES_PER_BLK = BT // S


def _gelu(v):
    return 0.5 * v * (1.0 + jax.lax.erf(v * 0.7071067811865476))


def _fused_body(x_ref, pe_w1_ref, pe_b1_ref, pe_w2_ref, pe_b2_ref,
                gate_w_ref, gate_b_ref, exp_w1_ref, exp_b1_ref,
                exp_w2_ref, exp_b2_ref, ln_g_ref, ln_b_ref,
                cls_w1_ref, cls_b1_ref, cls_w2_ref, cls_b2_ref,
                logits_ref, moe_ref, aux_ref,
                pooled_acc, imp_acc, cnt_acc):
    i = pl.program_id(0)

    @pl.when(i == 0)
    def _():
        imp_acc[...] = jnp.zeros_like(imp_acc)
        cnt_acc[...] = jnp.zeros_like(cnt_acc)

    xb = x_ref[...]
    h1 = _gelu(jnp.dot(xb, pe_w1_ref[...], preferred_element_type=jnp.float32)
               + pe_b1_ref[...])
    tok = (jnp.dot(h1, pe_w2_ref[...], preferred_element_type=jnp.float32)
           + pe_b2_ref[...])

    glog = (jnp.dot(tok, gate_w_ref[...], preferred_element_type=jnp.float32)
            + gate_b_ref[...])
    m = jnp.max(glog, axis=-1, keepdims=True)
    p = jnp.exp(glog - m)
    probs = p / jnp.sum(p, axis=-1, keepdims=True)
    gval = jnp.max(probs, axis=-1, keepdims=True)
    idx = jnp.argmax(probs, axis=-1).reshape(BT, 1)
    eids = jax.lax.broadcasted_iota(jnp.int32, (BT, E), 1)
    oh = (idx == eids).astype(jnp.float32)

    imp_acc[...] += jnp.sum(probs, axis=0, keepdims=True)
    cnt_acc[...] += jnp.sum(oh, axis=0, keepdims=True)

    combine = oh * gval
    tok16 = tok.astype(jnp.bfloat16)
    moe = jnp.dot(combine, exp_b2_ref[...],
                  preferred_element_type=jnp.float32)
    for e in range(E):
        he = _gelu(
            jnp.dot(tok16, exp_w1_ref[:, e * HID:(e + 1) * HID],
                    preferred_element_type=jnp.float32)
            + exp_b1_ref[:, e * HID:(e + 1) * HID])
        oe = jnp.dot(he.astype(jnp.bfloat16),
                     exp_w2_ref[e * HID:(e + 1) * HID, :],
                     preferred_element_type=jnp.float32)
        moe += combine[:, e:e + 1] * oe

    moe3 = moe.reshape(BATCHES_PER_BLK, S, D)
    moe_ref[...] = moe3
    pooled_acc[pl.ds(i * BATCHES_PER_BLK, BATCHES_PER_BLK), :] = (
        jnp.mean(moe3, axis=1))

    @pl.when(i == NBLK - 1)
    def _():
        pooled = pooled_acc[...]
        mu = jnp.mean(pooled, axis=-1, keepdims=True)
        var = jnp.mean((pooled - mu) ** 2, axis=-1, keepdims=True)
        ln = ((pooled - mu) / jnp.sqrt(var + 1e-5) * ln_g_ref[...]
              + ln_b_ref[...])
        c = _gelu(jnp.dot(ln, cls_w1_ref[...],
                          preferred_element_type=jnp.float32)
                  + cls_b1_ref[...])
        logits_ref[...] = (jnp.dot(c, cls_w2_ref[...],
                                   preferred_element_type=jnp.float32)
                           + cls_b2_ref[...])
        imp = imp_acc[...] / T
        load = cnt_acc[...] / T
        aux = E * jnp.sum(imp * load)
        aux_ref[...] = jnp.full((1, 128), aux, dtype=jnp.float32)


def kernel(x, pe_w1, pe_b1, pe_w2, pe_b2, gate_w, gate_b, exp_w1, exp_b1,
           exp_w2, exp_b2, ln_g, ln_b, cls_w1, cls_b1, cls_w2, cls_b2,
           interpret=False):
    xr = x.reshape(T, DIN)

    full = lambda shape: pl.BlockSpec(shape, lambda i: (0,) * len(shape))
    logits, moe_out, aux = pl.pallas_call(
        _fused_body,
        grid=(NBLK,),
        in_specs=[
            pl.BlockSpec((BT, DIN), lambda i: (i, 0)),
            full((DIN, D)),
            full((1, D)),
            full((D, D)),
            full((1, D)),
            full((D, E)),
            full((1, E)),
            full((D, E * HID)),
            full((1, E * HID)),
            full((E * HID, D)),
            full((E, D)),
            full((1, D)),
            full((1, D)),
            full((D, D)),
            full((1, D)),
            full((D, NCLS)),
            full((1, NCLS)),
        ],
        out_specs=[
            pl.BlockSpec((B, NCLS), lambda i: (0, 0)),
            pl.BlockSpec((BATCHES_PER_BLK, S, D), lambda i: (i, 0, 0)),
            pl.BlockSpec((1, 128), lambda i: (0, 0)),
        ],
        out_shape=[
            jax.ShapeDtypeStruct((B, NCLS), jnp.float32),
            jax.ShapeDtypeStruct((B, S, D), jnp.float32),
            jax.ShapeDtypeStruct((1, 128), jnp.float32),
        ],
        scratch_shapes=[
            pltpu.VMEM((B, D), jnp.float32),
            pltpu.VMEM((1, E), jnp.float32),
            pltpu.VMEM((1, E), jnp.float32),
        ],
        interpret=interpret,
    )(xr, pe_w1, pe_b1.reshape(1, D), pe_w2, pe_b2.reshape(1, D),
      gate_w, gate_b.reshape(1, E),
      exp_w1.transpose(1, 0, 2).reshape(D, E * HID).astype(jnp.bfloat16),
      exp_b1.reshape(1, E * HID),
      exp_w2.reshape(E * HID, D).astype(jnp.bfloat16),
      exp_b2,
      ln_g.reshape(1, D), ln_b.reshape(1, D), cls_w1,
      cls_b1.reshape(1, D), cls_w2, cls_b2.reshape(1, NCLS))
    return logits, moe_out, aux[0, 0]


# speedup vs baseline: 2.8682x; 1.4234x over previous
import functools

import jax
import jax.numpy as jnp
from jax.experimental import pallas as pl
from jax.experimental.pallas import tpu as pltpu

B, S, DIN, D, HID, E, NCLS = 64, 196, 768, 64, 256, 4, 10
T = B * S
BT = 784
NBLK = T // BT
BATCHES_PER_BLK = BT // S


def _gelu(v):
    return 0.5 * v * (1.0 + jax.lax.erf(v * 0.7071067811865476))


def _fused_body(x_ref, pe_w1_ref, pe_b1_ref, pe_w2_ref, pe_b2_ref,
                gate_w_ref, gate_b_ref, exp_w1_ref, exp_b1_ref,
                exp_w2_ref, exp_b2_ref, ln_g_ref, ln_b_ref,
                cls_w1_ref, cls_b1_ref, cls_w2_ref, cls_b2_ref,
                logits_ref, moe_ref, aux_ref,
                pooled_acc, imp_acc, cnt_acc):
    i = pl.program_id(0)

    @pl.when(i == 0)
    def _():
        imp_acc[...] = jnp.zeros_like(imp_acc)
        cnt_acc[...] = jnp.zeros_like(cnt_acc)

    xb = x_ref[...].reshape(BT, DIN)
    h1 = _gelu(jnp.dot(xb, pe_w1_ref[...], preferred_element_type=jnp.float32)
               + pe_b1_ref[...])
    tok = (jnp.dot(h1, pe_w2_ref[...], preferred_element_type=jnp.float32)
           + pe_b2_ref[...])

    glog = (jnp.dot(tok, gate_w_ref[...], preferred_element_type=jnp.float32)
            + gate_b_ref[...])
    m = jnp.max(glog, axis=-1, keepdims=True)
    p = jnp.exp(glog - m)
    probs = p / jnp.sum(p, axis=-1, keepdims=True)
    gval = jnp.max(probs, axis=-1, keepdims=True)
    idx = jnp.argmax(probs, axis=-1).reshape(BT, 1)
    eids = jax.lax.broadcasted_iota(jnp.int32, (BT, E), 1)
    oh = (idx == eids).astype(jnp.float32)

    imp_acc[...] += jnp.sum(probs, axis=0, keepdims=True)
    cnt_acc[...] += jnp.sum(oh, axis=0, keepdims=True)

    combine = oh * gval
    tok16 = tok.astype(jnp.bfloat16)
    moe = jnp.dot(combine, exp_b2_ref[...],
                  preferred_element_type=jnp.float32)
    for e in range(E):
        he = _gelu(
            jnp.dot(tok16, exp_w1_ref[:, e * HID:(e + 1) * HID],
                    preferred_element_type=jnp.float32)
            + exp_b1_ref[:, e * HID:(e + 1) * HID])
        oe = jnp.dot(he.astype(jnp.bfloat16),
                     exp_w2_ref[e * HID:(e + 1) * HID, :],
                     preferred_element_type=jnp.float32)
        moe += combine[:, e:e + 1] * oe

    moe3 = moe.reshape(BATCHES_PER_BLK, S, D)
    moe_ref[...] = moe3
    pooled_acc[pl.ds(i * BATCHES_PER_BLK, BATCHES_PER_BLK), :] = (
        jnp.mean(moe3, axis=1))

    @pl.when(i == NBLK - 1)
    def _():
        pooled = pooled_acc[...]
        mu = jnp.mean(pooled, axis=-1, keepdims=True)
        var = jnp.mean((pooled - mu) ** 2, axis=-1, keepdims=True)
        ln = ((pooled - mu) / jnp.sqrt(var + 1e-5) * ln_g_ref[...]
              + ln_b_ref[...])
        c = _gelu(jnp.dot(ln, cls_w1_ref[...],
                          preferred_element_type=jnp.float32)
                  + cls_b1_ref[...])
        logits_ref[...] = (jnp.dot(c, cls_w2_ref[...],
                                   preferred_element_type=jnp.float32)
                           + cls_b2_ref[...])
        imp = imp_acc[...] / T
        load = cnt_acc[...] / T
        aux = E * jnp.sum(imp * load)
        aux_ref[...] = jnp.full((1, 128), aux, dtype=jnp.float32)


def kernel(x, pe_w1, pe_b1, pe_w2, pe_b2, gate_w, gate_b, exp_w1, exp_b1,
           exp_w2, exp_b2, ln_g, ln_b, cls_w1, cls_b1, cls_w2, cls_b2,
           interpret=False):

    full = lambda shape: pl.BlockSpec(shape, lambda i: (0,) * len(shape))
    logits, moe_out, aux = pl.pallas_call(
        _fused_body,
        grid=(NBLK,),
        in_specs=[
            pl.BlockSpec((BATCHES_PER_BLK, S, DIN), lambda i: (i, 0, 0)),
            full((DIN, D)),
            full((1, D)),
            full((D, D)),
            full((1, D)),
            full((D, E)),
            full((1, E)),
            full((D, E * HID)),
            full((1, E * HID)),
            full((E * HID, D)),
            full((E, D)),
            full((1, D)),
            full((1, D)),
            full((D, D)),
            full((1, D)),
            full((D, NCLS)),
            full((1, NCLS)),
        ],
        out_specs=[
            pl.BlockSpec((B, NCLS), lambda i: (0, 0)),
            pl.BlockSpec((BATCHES_PER_BLK, S, D), lambda i: (i, 0, 0)),
            pl.BlockSpec((1, 128), lambda i: (0, 0)),
        ],
        out_shape=[
            jax.ShapeDtypeStruct((B, NCLS), jnp.float32),
            jax.ShapeDtypeStruct((B, S, D), jnp.float32),
            jax.ShapeDtypeStruct((1, 128), jnp.float32),
        ],
        scratch_shapes=[
            pltpu.VMEM((B, D), jnp.float32),
            pltpu.VMEM((1, E), jnp.float32),
            pltpu.VMEM((1, E), jnp.float32),
        ],
        interpret=interpret,
    )(x, pe_w1, pe_b1.reshape(1, D), pe_w2, pe_b2.reshape(1, D),
      gate_w, gate_b.reshape(1, E),
      exp_w1.transpose(1, 0, 2).reshape(D, E * HID).astype(jnp.bfloat16),
      exp_b1.reshape(1, E * HID),
      exp_w2.reshape(E * HID, D).astype(jnp.bfloat16),
      exp_b2,
      ln_g.reshape(1, D), ln_b.reshape(1, D), cls_w1,
      cls_b1.reshape(1, D), cls_w2, cls_b2.reshape(1, NCLS))
    return logits, moe_out, aux[0, 0]


# speedup vs baseline: 2.9259x; 1.0201x over previous
import functools

import jax
import jax.numpy as jnp
from jax.experimental import pallas as pl
from jax.experimental.pallas import tpu as pltpu

B, S, DIN, D, HID, E, NCLS = 64, 196, 768, 64, 256, 4, 10
T = B * S
BT = 1568
NBLK = T // BT
BATCHES_PER_BLK = BT // S


def _gelu(v):
    return 0.5 * v * (1.0 + jax.lax.erf(v * 0.7071067811865476))


def _fused_body(x_ref, pe_w1_ref, pe_b1_ref, pe_w2_ref, pe_b2_ref,
                gate_w_ref, gate_b_ref, exp_w1_ref, exp_b1_ref,
                exp_w2_ref, exp_b2_ref, ln_g_ref, ln_b_ref,
                cls_w1_ref, cls_b1_ref, cls_w2_ref, cls_b2_ref,
                logits_ref, moe_ref, aux_ref,
                pooled_acc, imp_acc, cnt_acc):
    i = pl.program_id(0)

    @pl.when(i == 0)
    def _():
        imp_acc[...] = jnp.zeros_like(imp_acc)
        cnt_acc[...] = jnp.zeros_like(cnt_acc)

    xb = x_ref[...].reshape(BT, DIN)
    h1 = _gelu(jnp.dot(xb, pe_w1_ref[...], preferred_element_type=jnp.float32)
               + pe_b1_ref[...])
    tok = (jnp.dot(h1, pe_w2_ref[...], preferred_element_type=jnp.float32)
           + pe_b2_ref[...])

    glog = (jnp.dot(tok, gate_w_ref[...], preferred_element_type=jnp.float32)
            + gate_b_ref[...])
    m = jnp.max(glog, axis=-1, keepdims=True)
    p = jnp.exp(glog - m)
    probs = p / jnp.sum(p, axis=-1, keepdims=True)
    gval = jnp.max(probs, axis=-1, keepdims=True)
    idx = jnp.argmax(probs, axis=-1).reshape(BT, 1)
    eids = jax.lax.broadcasted_iota(jnp.int32, (BT, E), 1)
    oh = (idx == eids).astype(jnp.float32)

    imp_acc[...] += jnp.sum(probs, axis=0, keepdims=True)
    cnt_acc[...] += jnp.sum(oh, axis=0, keepdims=True)

    combine = oh * gval
    tok16 = tok.astype(jnp.bfloat16)
    moe = jnp.dot(combine, exp_b2_ref[...],
                  preferred_element_type=jnp.float32)
    for e in range(E):
        he = _gelu(
            jnp.dot(tok16, exp_w1_ref[:, e * HID:(e + 1) * HID],
                    preferred_element_type=jnp.float32)
            + exp_b1_ref[:, e * HID:(e + 1) * HID])
        oe = jnp.dot(he.astype(jnp.bfloat16),
                     exp_w2_ref[e * HID:(e + 1) * HID, :],
                     preferred_element_type=jnp.float32)
        moe += combine[:, e:e + 1] * oe

    moe3 = moe.reshape(BATCHES_PER_BLK, S, D)
    moe_ref[...] = moe3
    pooled_acc[pl.ds(i * BATCHES_PER_BLK, BATCHES_PER_BLK), :] = (
        jnp.mean(moe3, axis=1))

    @pl.when(i == NBLK - 1)
    def _():
        pooled = pooled_acc[...]
        mu = jnp.mean(pooled, axis=-1, keepdims=True)
        var = jnp.mean((pooled - mu) ** 2, axis=-1, keepdims=True)
        ln = ((pooled - mu) / jnp.sqrt(var + 1e-5) * ln_g_ref[...]
              + ln_b_ref[...])
        c = _gelu(jnp.dot(ln, cls_w1_ref[...],
                          preferred_element_type=jnp.float32)
                  + cls_b1_ref[...])
        logits_ref[...] = (jnp.dot(c, cls_w2_ref[...],
                                   preferred_element_type=jnp.float32)
                           + cls_b2_ref[...])
        imp = imp_acc[...] / T
        load = cnt_acc[...] / T
        aux = E * jnp.sum(imp * load)
        aux_ref[...] = jnp.full((1, 128), aux, dtype=jnp.float32)


def kernel(x, pe_w1, pe_b1, pe_w2, pe_b2, gate_w, gate_b, exp_w1, exp_b1,
           exp_w2, exp_b2, ln_g, ln_b, cls_w1, cls_b1, cls_w2, cls_b2,
           interpret=False):

    full = lambda shape: pl.BlockSpec(shape, lambda i: (0,) * len(shape))
    logits, moe_out, aux = pl.pallas_call(
        _fused_body,
        grid=(NBLK,),
        in_specs=[
            pl.BlockSpec((BATCHES_PER_BLK, S, DIN), lambda i: (i, 0, 0)),
            full((DIN, D)),
            full((1, D)),
            full((D, D)),
            full((1, D)),
            full((D, E)),
            full((1, E)),
            full((D, E * HID)),
            full((1, E * HID)),
            full((E * HID, D)),
            full((E, D)),
            full((1, D)),
            full((1, D)),
            full((D, D)),
            full((1, D)),
            full((D, NCLS)),
            full((1, NCLS)),
        ],
        out_specs=[
            pl.BlockSpec((B, NCLS), lambda i: (0, 0)),
            pl.BlockSpec((BATCHES_PER_BLK, S, D), lambda i: (i, 0, 0)),
            pl.BlockSpec((1, 128), lambda i: (0, 0)),
        ],
        out_shape=[
            jax.ShapeDtypeStruct((B, NCLS), jnp.float32),
            jax.ShapeDtypeStruct((B, S, D), jnp.float32),
            jax.ShapeDtypeStruct((1, 128), jnp.float32),
        ],
        scratch_shapes=[
            pltpu.VMEM((B, D), jnp.float32),
            pltpu.VMEM((1, E), jnp.float32),
            pltpu.VMEM((1, E), jnp.float32),
        ],
        interpret=interpret,
    )(x, pe_w1, pe_b1.reshape(1, D), pe_w2, pe_b2.reshape(1, D),
      gate_w, gate_b.reshape(1, E),
      exp_w1.transpose(1, 0, 2).reshape(D, E * HID).astype(jnp.bfloat16),
      exp_b1.reshape(1, E * HID),
      exp_w2.reshape(E * HID, D).astype(jnp.bfloat16),
      exp_b2,
      ln_g.reshape(1, D), ln_b.reshape(1, D), cls_w1,
      cls_b1.reshape(1, D), cls_w2, cls_b2.reshape(1, NCLS))
    return logits, moe_out, aux[0, 0]


# speedup vs baseline: 4.8770x; 1.6668x over previous
import jax
import jax.numpy as jnp
from jax.experimental import pallas as pl
from jax.experimental.pallas import tpu as pltpu

B, S, DIN, D, HID, E, NCLS = 64, 196, 768, 64, 256, 4, 10
T = B * S
SB = 28
NBLK = S // SB
BT = SB * B


def _gelu(v):
    return 0.5 * v * (1.0 + jax.lax.erf(v * 0.7071067811865476))


def _fused_body(x_ref, pe_w1t_ref, pe_b1_ref, pe_w2_ref, pe_b2_ref,
                gate_w_ref, gate_b_ref, exp_w1_ref, exp_b1_ref,
                exp_w2_ref, exp_b2_ref, ln_g_ref, ln_b_ref,
                cls_w1_ref, cls_b1_ref, cls_w2_ref, cls_b2_ref,
                logits_ref, moe_ref, aux_ref,
                pooled_acc, imp_acc, cnt_acc):
    i = pl.program_id(0)

    @pl.when(i == 0)
    def _():
        pooled_acc[...] = jnp.zeros_like(pooled_acc)
        imp_acc[...] = jnp.zeros_like(imp_acc)
        cnt_acc[...] = jnp.zeros_like(cnt_acc)

    xb = x_ref[...].reshape(BT, DIN)
    h1 = _gelu(
        jax.lax.dot_general(xb, pe_w1t_ref[...],
                            (((1,), (1,)), ((), ())),
                            preferred_element_type=jnp.float32)
        + pe_b1_ref[...])
    tok = (jnp.dot(h1, pe_w2_ref[...], preferred_element_type=jnp.float32)
           + pe_b2_ref[...])

    glog = (jnp.dot(tok, gate_w_ref[...], preferred_element_type=jnp.float32)
            + gate_b_ref[...])
    m = jnp.max(glog, axis=-1, keepdims=True)
    p = jnp.exp(glog - m)
    probs = p / jnp.sum(p, axis=-1, keepdims=True)
    gval = jnp.max(probs, axis=-1, keepdims=True)
    idx = jnp.argmax(probs, axis=-1).reshape(BT, 1)
    eids = jax.lax.broadcasted_iota(jnp.int32, (BT, E), 1)
    oh = (idx == eids).astype(jnp.float32)

    imp_acc[...] += jnp.sum(probs, axis=0, keepdims=True)
    cnt_acc[...] += jnp.sum(oh, axis=0, keepdims=True)

    combine = oh * gval
    tok16 = tok.astype(jnp.bfloat16)
    moe = jnp.dot(combine, exp_b2_ref[...],
                  preferred_element_type=jnp.float32)
    for e in range(E):
        he = _gelu(
            jnp.dot(tok16, exp_w1_ref[:, e * HID:(e + 1) * HID],
                    preferred_element_type=jnp.float32)
            + exp_b1_ref[:, e * HID:(e + 1) * HID])
        oe = jnp.dot(he.astype(jnp.bfloat16),
                     exp_w2_ref[e * HID:(e + 1) * HID, :],
                     preferred_element_type=jnp.float32)
        moe += combine[:, e:e + 1] * oe

    moe3 = moe.reshape(SB, B, D)
    moe_ref[...] = moe3
    pooled_acc[...] += jnp.sum(moe3, axis=0)

    @pl.when(i == NBLK - 1)
    def _():
        pooled = pooled_acc[...] / S
        mu = jnp.mean(pooled, axis=-1, keepdims=True)
        var = jnp.mean((pooled - mu) ** 2, axis=-1, keepdims=True)
        ln = ((pooled - mu) / jnp.sqrt(var + 1e-5) * ln_g_ref[...]
              + ln_b_ref[...])
        c = _gelu(jnp.dot(ln, cls_w1_ref[...],
                          preferred_element_type=jnp.float32)
                  + cls_b1_ref[...])
        logits_ref[...] = (jnp.dot(c, cls_w2_ref[...],
                                   preferred_element_type=jnp.float32)
                           + cls_b2_ref[...])
        imp = imp_acc[...] / T
        load = cnt_acc[...] / T
        aux = E * jnp.sum(imp * load)
        aux_ref[...] = jnp.full((1, 128), aux, dtype=jnp.float32)


def kernel(x, pe_w1, pe_b1, pe_w2, pe_b2, gate_w, gate_b, exp_w1, exp_b1,
           exp_w2, exp_b2, ln_g, ln_b, cls_w1, cls_b1, cls_w2, cls_b2,
           interpret=False):
    xt = jnp.transpose(x, (1, 0, 2))

    full = lambda shape: pl.BlockSpec(shape, lambda i: (0,) * len(shape))
    logits, moe_s, aux = pl.pallas_call(
        _fused_body,
        grid=(NBLK,),
        in_specs=[
            pl.BlockSpec((SB, B, DIN), lambda i: (i, 0, 0)),
            full((D, DIN)),
            full((1, D)),
            full((D, D)),
            full((1, D)),
            full((D, E)),
            full((1, E)),
            full((D, E * HID)),
            full((1, E * HID)),
            full((E * HID, D)),
            full((E, D)),
            full((1, D)),
            full((1, D)),
            full((D, D)),
            full((1, D)),
            full((D, NCLS)),
            full((1, NCLS)),
        ],
        out_specs=[
            pl.BlockSpec((B, NCLS), lambda i: (0, 0)),
            pl.BlockSpec((SB, B, D), lambda i: (i, 0, 0)),
            pl.BlockSpec((1, 128), lambda i: (0, 0)),
        ],
        out_shape=[
            jax.ShapeDtypeStruct((B, NCLS), jnp.float32),
            jax.ShapeDtypeStruct((S, B, D), jnp.float32),
            jax.ShapeDtypeStruct((1, 128), jnp.float32),
        ],
        scratch_shapes=[
            pltpu.VMEM((B, D), jnp.float32),
            pltpu.VMEM((1, E), jnp.float32),
            pltpu.VMEM((1, E), jnp.float32),
        ],
        interpret=interpret,
    )(xt, pe_w1.T, pe_b1.reshape(1, D), pe_w2, pe_b2.reshape(1, D),
      gate_w, gate_b.reshape(1, E),
      exp_w1.transpose(1, 0, 2).reshape(D, E * HID).astype(jnp.bfloat16),
      exp_b1.reshape(1, E * HID),
      exp_w2.reshape(E * HID, D).astype(jnp.bfloat16),
      exp_b2,
      ln_g.reshape(1, D), ln_b.reshape(1, D), cls_w1,
      cls_b1.reshape(1, D), cls_w2, cls_b2.reshape(1, NCLS))
    moe_out = jnp.transpose(moe_s, (1, 0, 2))
    return logits, moe_out, aux[0, 0]


# speedup vs baseline: 5.4402x; 1.1155x over previous
import jax
import jax.numpy as jnp
from jax.experimental import pallas as pl
from jax.experimental.pallas import tpu as pltpu

B, S, DIN, D, HID, E, NCLS = 64, 196, 768, 64, 256, 4, 10
T = B * S
SB = 28
NBLK = S // SB
BT = SB * B


def _gelu(v):
    return 0.5 * v * (1.0 + jax.lax.erf(v * 0.7071067811865476))


def _fused_body(x_ref, pe_w1t_ref, pe_b1_ref, pe_w2_ref, pe_b2_ref,
                gate_w_ref, gate_b_ref, exp_w1_ref, exp_b1_ref,
                exp_w2_ref, exp_b2_ref, ln_g_ref, ln_b_ref,
                cls_w1_ref, cls_b1_ref, cls_w2_ref, cls_b2_ref,
                logits_ref, moe_ref, aux_ref,
                pooled_acc, imp_acc, cnt_acc):
    i = pl.program_id(0)

    @pl.when(i == 0)
    def _():
        pooled_acc[...] = jnp.zeros_like(pooled_acc)
        imp_acc[...] = jnp.zeros_like(imp_acc)
        cnt_acc[...] = jnp.zeros_like(cnt_acc)

    xb = x_ref[...].reshape(BT, DIN)
    h1 = _gelu(
        jax.lax.dot_general(xb, pe_w1t_ref[...],
                            (((1,), (1,)), ((), ())),
                            preferred_element_type=jnp.float32)
        + pe_b1_ref[...])
    tok = (jnp.dot(h1, pe_w2_ref[...], preferred_element_type=jnp.float32)
           + pe_b2_ref[...])

    glog = (jax.lax.dot_general(tok, gate_w_ref[...],
                                (((1,), (1,)), ((), ())),
                                preferred_element_type=jnp.float32)
            + gate_b_ref[...])
    m = jnp.max(glog, axis=-1, keepdims=True)
    p = jnp.exp(glog - m)
    probs = p / jnp.sum(p, axis=-1, keepdims=True)
    gval = jnp.max(probs, axis=-1, keepdims=True)
    idx = jnp.argmax(probs, axis=-1).reshape(BT, 1)
    eids = jax.lax.broadcasted_iota(jnp.int32, (BT, E), 1)
    oh = (idx == eids).astype(jnp.float32)

    imp_acc[...] += jnp.sum(probs, axis=0, keepdims=True)
    cnt_acc[...] += jnp.sum(oh, axis=0, keepdims=True)

    combine = oh * gval
    tok16 = tok.astype(jnp.bfloat16)
    moe = jnp.dot(combine, exp_b2_ref[...],
                  preferred_element_type=jnp.float32)
    for e in range(E):
        he = _gelu(
            jnp.dot(tok16, exp_w1_ref[e].astype(jnp.bfloat16),
                    preferred_element_type=jnp.float32)
            + exp_b1_ref[e][None, :])
        oe = jnp.dot(he.astype(jnp.bfloat16),
                     exp_w2_ref[e].astype(jnp.bfloat16),
                     preferred_element_type=jnp.float32)
        moe += combine[:, e:e + 1] * oe

    moe3 = moe.reshape(SB, B, D)
    moe_ref[...] = moe3
    pooled_acc[...] += jnp.sum(moe3, axis=0)

    @pl.when(i == NBLK - 1)
    def _():
        pooled = pooled_acc[...] / S
        mu = jnp.mean(pooled, axis=-1, keepdims=True)
        var = jnp.mean((pooled - mu) ** 2, axis=-1, keepdims=True)
        ln = ((pooled - mu) / jnp.sqrt(var + 1e-5) * ln_g_ref[...]
              + ln_b_ref[...])
        c = _gelu(jnp.dot(ln, cls_w1_ref[...],
                          preferred_element_type=jnp.float32)
                  + cls_b1_ref[...])
        logits_ref[...] = (jax.lax.dot_general(
            cls_w2_ref[...], c, (((1,), (1,)), ((), ())),
            preferred_element_type=jnp.float32) + cls_b2_ref[...])
        imp = imp_acc[...] / T
        load = cnt_acc[...] / T
        aux = E * jnp.sum(imp * load)
        aux_ref[...] = jnp.full((1, 128), aux, dtype=jnp.float32)


def kernel(x, pe_w1, pe_b1, pe_w2, pe_b2, gate_w, gate_b, exp_w1, exp_b1,
           exp_w2, exp_b2, ln_g, ln_b, cls_w1, cls_b1, cls_w2, cls_b2,
           interpret=False):
    xt = jnp.transpose(x, (1, 0, 2))

    full = lambda shape: pl.BlockSpec(shape, lambda i: (0,) * len(shape))
    logits, moe_s, aux = pl.pallas_call(
        _fused_body,
        grid=(NBLK,),
        in_specs=[
            pl.BlockSpec((SB, B, DIN), lambda i: (i, 0, 0)),
            full((D, DIN)),
            full((1, D)),
            full((D, D)),
            full((1, D)),
            full((E, D)),
            full((1, E)),
            full((E, D, HID)),
            full((E, HID)),
            full((E, HID, D)),
            full((E, D)),
            full((1, D)),
            full((1, D)),
            full((D, D)),
            full((1, D)),
            full((NCLS, D)),
            full((NCLS, 1)),
        ],
        out_specs=[
            pl.BlockSpec((NCLS, B), lambda i: (0, 0)),
            pl.BlockSpec((SB, B, D), lambda i: (i, 0, 0)),
            pl.BlockSpec((1, 128), lambda i: (0, 0)),
        ],
        out_shape=[
            jax.ShapeDtypeStruct((NCLS, B), jnp.float32),
            jax.ShapeDtypeStruct((S, B, D), jnp.float32),
            jax.ShapeDtypeStruct((1, 128), jnp.float32),
        ],
        scratch_shapes=[
            pltpu.VMEM((B, D), jnp.float32),
            pltpu.VMEM((1, E), jnp.float32),
            pltpu.VMEM((1, E), jnp.float32),
        ],
        interpret=interpret,
    )(xt, pe_w1.T, pe_b1.reshape(1, D), pe_w2, pe_b2.reshape(1, D),
      gate_w.T, gate_b.reshape(1, E),
      exp_w1, exp_b1, exp_w2, exp_b2,
      ln_g.reshape(1, D), ln_b.reshape(1, D), cls_w1,
      cls_b1.reshape(1, D), cls_w2.T, cls_b2.reshape(NCLS, 1))
    moe_out = jnp.transpose(moe_s, (1, 0, 2))
    return logits.T, moe_out, aux[0, 0]
